# baseline (speedup 1.0000x reference)
import jax
import jax.numpy as jnp
from jax import lax
from jax.experimental import pallas as pl
from jax.experimental.pallas import tpu as pltpu
from jax.experimental.pallas import tpu_sc as plsc

_D = 64
_NC, _NS = 2, 16
_NW = _NC * _NS
_BB = 128
_NG = 4
_DRAIN = 2


def _body(idxT_hbm, table_hbm, out_hbm, idx_v, t_v, g0, g1, g2, g3, *sems):
    h_tot, b_tot = idxT_hbm.shape
    g_v = [g0, g1, g2, g3]
    gsems, osems = sems[:_NG], sems[_NG:]
    wid = lax.axis_index("s") * _NC + lax.axis_index("c")
    blocks_per_w = b_tot // _BB // _NW
    pairs = blocks_per_w * h_tot
    c0 = wid * blocks_per_w

    iota16 = lax.iota(jnp.int32, 16)
    rowidx = [iota16 + (l0 * 16) for l0 in range(8)]

    def gather_copy(i, u, sem):
        cl, hh = i // h_tot, lax.rem(i, h_tot)
        return pltpu.make_async_copy(
            table_hbm.at[idx_v.at[lax.rem(cl, 2), hh]], g_v[u], sem)

    def out_copy(k, u2, sem):
        cl, hh = k // h_tot, lax.rem(k, h_tot)
        return pltpu.make_async_copy(
            t_v.at[u2], out_hbm.at[hh, :, c0 + cl], sem)

    def step(t, u, carry):
        i = t * _NG + u

        @pl.when(i < pairs)
        def _():
            cl = i // h_tot

            @pl.when(lax.rem(i, h_tot) == 0)
            def _():
                pltpu.sync_copy(
                    idxT_hbm.at[:, pl.ds((c0 + cl) * _BB, _BB)],
                    idx_v.at[lax.rem(cl, 2)])
            gather_copy(i, u, gsems[u]).start()

        k = i - _DRAIN
        up, u2 = (u - _DRAIN) % _NG, u % 2

        @pl.when((k >= 0) & (k < pairs))
        def _():
            gather_copy(0, up, gsems[up]).wait()

            @pl.when(k >= 2)
            def _():
                out_copy(0, u2, osems[u2]).wait()
            gsrc = g_v[up]

            def transp(d, cr):
                di, dm = d // 8, lax.rem(d, 8)
                col16 = jnp.full((16,), d, jnp.int32)
                for l0 in range(8):
                    v = plsc.load_gather(gsrc, [rowidx[l0], col16])
                    t_v[u2, di, dm, pl.ds(l0 * 16, 16)] = v
                return cr

            lax.fori_loop(0, _D, transp, 0, unroll=8)
            out_copy(k, u2, osems[u2]).start()
        return carry

    outer = -(-(pairs + _DRAIN) // _NG)
    lax.fori_loop(
        0, outer,
        lambda t, cr: [step(t, u, cr) for u in range(_NG)][-1], 0)
    for u2 in range(2):
        out_copy(0, u2, osems[u2]).wait()


def kernel(input, weight):
    b, h = input.shape
    f = pl.kernel(
        _body,
        out_type=jax.ShapeDtypeStruct((h, 8, b // _BB, 8, _BB), jnp.float32),
        mesh=plsc.VectorSubcoreMesh(core_axis_name="c", subcore_axis_name="s"),
        scratch_types=(
            [pltpu.VMEM((2, h, _BB), jnp.int32),
             pltpu.VMEM((2, 8, 8, _BB), jnp.float32)]
            + [pltpu.VMEM((_BB, _D), jnp.float32)] * _NG
            + [pltpu.SemaphoreType.DMA] * (_NG + 2)
        ),
        compiler_params=pltpu.CompilerParams(
            use_tc_tiling_on_sc=False, needs_layout_passes=False),
    )
    out5 = f(input.T.astype(jnp.int32), weight)
    return out5.transpose(2, 4, 0, 1, 3).reshape(b, h, _D)


# speedup vs baseline: 1.4394x; 1.4394x over previous
import jax
import jax.numpy as jnp
from jax import lax
from jax.experimental import pallas as pl
from jax.experimental.pallas import tpu as pltpu
from jax.experimental.pallas import tpu_sc as plsc

_D = 64
_NC, _NS = 2, 16
_NW = _NC * _NS
_BB = 128
_NG = 4
_DRAIN = 2


def _body(idxT_hbm, table_hbm, out_hbm, idx_v, t_v, g0, g1, g2, g3, *sems):
    h_tot, b_tot = idxT_hbm.shape
    g_v = [g0, g1, g2, g3]
    gsems, osems = sems[:_NG], sems[_NG:]
    wid = lax.axis_index("s") * _NC + lax.axis_index("c")
    blocks_per_w = b_tot // _BB // _NW
    pairs = blocks_per_w * h_tot
    c0 = wid * blocks_per_w

    iota16 = lax.iota(jnp.int32, 16)
    rowidx = [iota16 + (l0 * 16) for l0 in range(8)]

    def gather_copy(i, u, sem):
        cl, hh = i // h_tot, lax.rem(i, h_tot)
        return pltpu.make_async_copy(
            table_hbm.at[idx_v.at[lax.rem(cl, 2), hh]], g_v[u], sem)

    def out_copy(k, u2, sem):
        cl, hh = k // h_tot, lax.rem(k, h_tot)
        return pltpu.make_async_copy(
            t_v.at[u2], out_hbm.at[hh, :, c0 + cl], sem)

    def step(t, u, carry):
        i = t * _NG + u

        @pl.when(i < pairs)
        def _():
            cl = i // h_tot

            @pl.when(lax.rem(i, h_tot) == 0)
            def _():
                pltpu.sync_copy(
                    idxT_hbm.at[:, pl.ds((c0 + cl) * _BB, _BB)],
                    idx_v.at[lax.rem(cl, 2)])
            gather_copy(i, u, gsems[u]).start()

        k = i - _DRAIN
        up, u2 = (u - _DRAIN) % _NG, u % 2

        @pl.when((k >= 0) & (k < pairs))
        def _():
            gather_copy(0, up, gsems[up]).wait()

            @pl.when(k >= 2)
            def _():
                out_copy(0, u2, osems[u2]).wait()
            gsrc = g_v[up]

            @plsc.parallel_loop(0, _D, unroll=8)
            def _(d):
                di, dm = d // 8, lax.rem(d, 8)
                col16 = jnp.full((16,), d, jnp.int32)
                for l0 in range(8):
                    v = plsc.load_gather(gsrc, [rowidx[l0], col16])
                    t_v[u2, di, dm, pl.ds(l0 * 16, 16)] = v
            out_copy(k, u2, osems[u2]).start()
        return carry

    outer = -(-(pairs + _DRAIN) // _NG)
    lax.fori_loop(
        0, outer,
        lambda t, cr: [step(t, u, cr) for u in range(_NG)][-1], 0)
    for u2 in range(2):
        out_copy(0, u2, osems[u2]).wait()


def kernel(input, weight):
    b, h = input.shape
    f = pl.kernel(
        _body,
        out_type=jax.ShapeDtypeStruct((h, 8, b // _BB, 8, _BB), jnp.float32),
        mesh=plsc.VectorSubcoreMesh(core_axis_name="c", subcore_axis_name="s"),
        scratch_types=(
            [pltpu.VMEM((2, h, _BB), jnp.int32),
             pltpu.VMEM((2, 8, 8, _BB), jnp.float32)]
            + [pltpu.VMEM((_BB, _D), jnp.float32)] * _NG
            + [pltpu.SemaphoreType.DMA] * (_NG + 2)
        ),
        compiler_params=pltpu.CompilerParams(
            use_tc_tiling_on_sc=False, needs_layout_passes=False),
    )
    out5 = f(input.T.astype(jnp.int32), weight)
    return out5.transpose(2, 4, 0, 1, 3).reshape(b, h, _D)


# speedup vs baseline: 1.5130x; 1.0511x over previous
import jax
import jax.numpy as jnp
from jax import lax
from jax.experimental import pallas as pl
from jax.experimental.pallas import tpu as pltpu
from jax.experimental.pallas import tpu_sc as plsc

_D = 64
_NC, _NS = 2, 16
_NW = _NC * _NS
_BB = 128
_NG = 4
_DRAIN = 2


def _body(idxT_hbm, table_hbm, out_hbm, idx_v, t_v, g0, g1, g2, g3, *sems):
    h_tot, b_tot = idxT_hbm.shape
    g_v = [g0, g1, g2, g3]
    gsems, osems = sems[:_NG], sems[_NG:]
    wid = lax.axis_index("s") * _NC + lax.axis_index("c")
    blocks_per_w = b_tot // _BB // _NW
    pairs = blocks_per_w * h_tot
    c0 = wid * blocks_per_w

    iota16 = lax.iota(jnp.int32, 16)
    rowidx = [iota16 + (l0 * 16) for l0 in range(8)]

    def gather_copy(i, u, sem):
        cl, hh = i // h_tot, lax.rem(i, h_tot)
        return pltpu.make_async_copy(
            table_hbm.at[idx_v.at[lax.rem(cl, 2), hh]], g_v[u], sem)

    def out_copy(k, u2, sem):
        cl, hh = k // h_tot, lax.rem(k, h_tot)
        return pltpu.make_async_copy(
            t_v.at[u2], out_hbm.at[hh, :, c0 + cl], sem)

    def step(t, u, carry):
        i = t * _NG + u

        @pl.when(i < pairs)
        def _():
            cl = i // h_tot

            @pl.when(lax.rem(i, h_tot) == 0)
            def _():
                pltpu.sync_copy(
                    idxT_hbm.at[:, pl.ds((c0 + cl) * _BB, _BB)],
                    idx_v.at[lax.rem(cl, 2)])
            gather_copy(i, u, gsems[u]).start()

        k = i - _DRAIN
        up, u2 = (u - _DRAIN) % _NG, u % 2

        @pl.when((k >= 0) & (k < pairs))
        def _():
            gather_copy(0, up, gsems[up]).wait()

            @pl.when(k >= 2)
            def _():
                out_copy(0, u2, osems[u2]).wait()
            gsrc = g_v[up]

            @plsc.parallel_loop(0, _D, unroll=8)
            def _(d):
                di, dm = d // 8, lax.rem(d, 8)
                col16 = jnp.full((16,), d, jnp.int32)
                for l0 in range(8):
                    v = plsc.load_gather(gsrc, [rowidx[l0], col16])
                    t_v[u2, di, dm, pl.ds(l0 * 16, 16)] = v
            out_copy(k, u2, osems[u2]).start()
        return carry

    outer = -(-(pairs + _DRAIN) // _NG)
    lax.fori_loop(
        0, outer,
        lambda t, cr: [step(t, u, cr) for u in range(_NG)][-1], 0)
    for u2 in range(2):
        out_copy(0, u2, osems[u2]).wait()


def kernel(input, weight):
    b, h = input.shape
    f = pl.kernel(
        _body,
        out_type=jax.ShapeDtypeStruct((h, 8, b // _BB, 8, _BB), jnp.float32),
        mesh=plsc.VectorSubcoreMesh(core_axis_name="c", subcore_axis_name="s"),
        scratch_types=(
            [pltpu.VMEM((2, h, _BB), jnp.int32),
             pltpu.VMEM((2, 8, 8, _BB), jnp.float32)]
            + [pltpu.VMEM((_BB, _BB), jnp.float32)] * _NG
            + [pltpu.SemaphoreType.DMA] * (_NG + 2)
        ),
        compiler_params=pltpu.CompilerParams(
            use_tc_tiling_on_sc=False, needs_layout_passes=False),
    )
    out5 = f(input.T.astype(jnp.int32), jnp.pad(weight, ((0, 0), (0, _BB - _D))))
    return out5.transpose(2, 4, 0, 1, 3).reshape(b, h, _D)
